# Initial kernel scaffold; baseline (speedup 1.0000x reference)
import jax
import jax.numpy as jnp
from jax.experimental import pallas as pl


def kernel(node_features, edge_index, edge_attr, subject_features, W_in, b_in, W_ee, b_ee, W1, b1, gamma, beta, W2, b2, W_ext, b_ext, W_pred, b_pred):
    raise NotImplementedError("write your pallas kernel here")


# speedup vs baseline: 3.4804x; 3.4804x over previous
import functools

import jax
import jax.numpy as jnp
from jax import lax
from jax.experimental import pallas as pl
from jax.experimental.pallas import tpu as pltpu
from jax.experimental.pallas import tpu_sc as plsc

B = 2
N = 10000
E = 160000
EMB = 128
H2 = 256
OUT = 128
EXT = 64
NPAD = 10240
NT = 16
NCHUNK = 125
CW = 80
SW = 16
RPT = NPAD // NT
NBLK = NPAD // CW


def _sc_mesh():
    return plsc.VectorSubcoreMesh(core_axis_name="c", subcore_axis_name="s")


def _sdeg_body(rows_hbm, dst_hbm, out_hbm, dst_v, row_buf, acc, sem):
    c = lax.axis_index("c")
    s = lax.axis_index("s")
    w = c * NT + s
    z = jnp.zeros((16,), jnp.float32)
    for i in range(CW):
        row_buf[i, :] = z
    for k in range(RPT // CW):
        pltpu.sync_copy(row_buf, acc.at[pl.ds(s * RPT + k * CW, CW)])
    pltpu.sync_copy(dst_hbm.at[w], dst_v)
    plsc.subcore_barrier()

    def body(j, carry):
        pltpu.async_copy(rows_hbm.at[w * NCHUNK + j], row_buf, sem).wait()
        pltpu.sync_copy(row_buf, acc.at[dst_v.at[j]], add=True)
        return carry

    lax.fori_loop(0, NCHUNK, body, 0)
    plsc.subcore_barrier()
    pltpu.sync_copy(acc.at[pl.ds(s * RPT, RPT)],
                    out_hbm.at[pl.ds(c * NPAD + s * RPT, RPT)])


def _sdeg(rows, dst_l):
    f = functools.partial(
        pl.kernel,
        out_type=jax.ShapeDtypeStruct((B * NPAD, SW), jnp.float32),
        mesh=_sc_mesh(),
        scratch_types=[
            pltpu.VMEM((NCHUNK, CW), jnp.int32),
            pltpu.VMEM((CW, SW), jnp.float32),
            pltpu.VMEM_SHARED((NPAD, SW), jnp.float32),
            pltpu.SemaphoreType.DMA,
        ],
    )(_sdeg_body)
    return f(rows, dst_l)


def _aggr_body(x_hbm, src_hbm, dst_hbm, out_hbm, src_v, dst_v, buf, acc, sem):
    c = lax.axis_index("c")
    s = lax.axis_index("s")
    w = c * NT + s
    z = jnp.zeros((16,), jnp.float32)

    def zrow(i, carry):
        for k in range(EMB // 16):
            buf[i, pl.ds(k * 16, 16)] = z
        return carry

    lax.fori_loop(0, CW, zrow, 0)
    for k in range(RPT // CW):
        pltpu.sync_copy(buf, acc.at[pl.ds(s * RPT + k * CW, CW)])
    pltpu.sync_copy(src_hbm.at[w], src_v)
    pltpu.sync_copy(dst_hbm.at[w], dst_v)
    plsc.subcore_barrier()

    def body(j, carry):
        pltpu.async_copy(x_hbm.at[src_v.at[j]], buf, sem).wait()
        pltpu.sync_copy(buf, acc.at[dst_v.at[j]], add=True)
        return carry

    lax.fori_loop(0, NCHUNK, body, 0)
    plsc.subcore_barrier()
    pltpu.sync_copy(acc.at[pl.ds(s * RPT, RPT)],
                    out_hbm.at[pl.ds(c * NPAD + s * RPT, RPT)])


def _aggr(x_flat, src_g, dst_l):
    f = functools.partial(
        pl.kernel,
        out_type=jax.ShapeDtypeStruct((B * NPAD, EMB), jnp.float32),
        mesh=_sc_mesh(),
        scratch_types=[
            pltpu.VMEM((NCHUNK, CW), jnp.int32),
            pltpu.VMEM((NCHUNK, CW), jnp.int32),
            pltpu.VMEM((CW, EMB), jnp.float32),
            pltpu.VMEM_SHARED((NPAD, EMB), jnp.float32),
            pltpu.SemaphoreType.DMA,
        ],
    )(_aggr_body)
    return f(x_flat, src_g, dst_l)


def _wc_call(W_ee, b_ee, W1):
    def body(we_ref, be_ref, w1_ref, o_ref):
        w1b = w1_ref[0, EMB:, :]
        r0 = we_ref[0] @ w1b
        r1 = be_ref[0] @ w1b
        o_ref[...] = jnp.concatenate(
            [r0, r1, jnp.zeros((SW - 2, H2), jnp.float32)], 0)[None]

    return pl.pallas_call(
        body,
        grid=(3,),
        in_specs=[
            pl.BlockSpec((1, 1, EMB), lambda l: (l, 0, 0)),
            pl.BlockSpec((1, 1, EMB), lambda l: (l, 0, 0)),
            pl.BlockSpec((1, 2 * EMB, H2), lambda l: (l, 0, 0)),
        ],
        out_specs=pl.BlockSpec((1, SW, H2), lambda l: (l, 0, 0)),
        out_shape=jax.ShapeDtypeStruct((3, SW, H2), jnp.float32),
    )(W_ee, b_ee.reshape(3, 1, EMB), W1)


def _x0_call(nf_pad, W_in_pad, b_in_row):
    def body(nf_ref, w_ref, b_ref, o_ref):
        o_ref[...] = nf_ref[0] @ w_ref[...] + b_ref[...]

    return pl.pallas_call(
        body,
        grid=(B, N // CW),
        in_specs=[
            pl.BlockSpec((1, CW, 24), lambda b, j: (b, j, 0)),
            pl.BlockSpec((24, EMB), lambda b, j: (0, 0)),
            pl.BlockSpec((1, EMB), lambda b, j: (0, 0)),
        ],
        out_specs=pl.BlockSpec((CW, EMB), lambda b, j: (b * NBLK + j, 0)),
        out_shape=jax.ShapeDtypeStruct((B * NPAD, EMB), jnp.float32),
    )(nf_pad, W_in_pad, b_in_row)


def _tstats_call(aggr_flat, S_flat, W1t, Wc_l, b1_row):
    def body(a_ref, s_ref, w_ref, wc_ref, b_ref, t_ref, st_ref):
        j = pl.program_id(1)
        t = a_ref[...] @ w_ref[...] + s_ref[...] @ wc_ref[...] + b_ref[...]
        t_ref[...] = t[None]
        p = jnp.concatenate(
            [jnp.sum(t, 0, keepdims=True),
             jnp.sum(t * t, 0, keepdims=True),
             jnp.zeros((6, H2), jnp.float32)], 0)[None]

        @pl.when(j == 0)
        def _():
            st_ref[...] = p

        @pl.when(j > 0)
        def _():
            st_ref[...] += p

    return pl.pallas_call(
        body,
        grid=(B, N // CW),
        in_specs=[
            pl.BlockSpec((CW, EMB), lambda b, j: (b * NBLK + j, 0)),
            pl.BlockSpec((CW, SW), lambda b, j: (b * NBLK + j, 0)),
            pl.BlockSpec((EMB, H2), lambda b, j: (0, 0)),
            pl.BlockSpec((SW, H2), lambda b, j: (0, 0)),
            pl.BlockSpec((1, H2), lambda b, j: (0, 0)),
        ],
        out_specs=[
            pl.BlockSpec((1, CW, H2), lambda b, j: (b, j, 0)),
            pl.BlockSpec((1, 8, H2), lambda b, j: (b, 0, 0)),
        ],
        out_shape=[
            jax.ShapeDtypeStruct((B, N, H2), jnp.float32),
            jax.ShapeDtypeStruct((B, 8, H2), jnp.float32),
        ],
    )(aggr_flat, S_flat, W1t, Wc_l, b1_row)


def _update_call(t, stats, gamma_row, beta_row, W2_l, b2_row, last):
    def body(t_ref, st_ref, g_ref, be_ref, w_ref, b_ref, x_ref, *rest):
        mean = st_ref[0, 0, :] * (1.0 / N)
        var = st_ref[0, 1, :] * (1.0 / N) - mean * mean
        inv = lax.rsqrt(var + 1e-5)
        tn = (t_ref[0] - mean) * (inv * g_ref[0]) + be_ref[0]
        r = jnp.maximum(tn, 0.0)
        hb = r @ w_ref[...] + b_ref[...]
        if not last:
            hb = jnp.maximum(hb, 0.0)
        x_ref[...] = hb
        if last:
            p_ref = rest[0]
            j = pl.program_id(1)
            p = jnp.concatenate(
                [jnp.sum(hb, 0, keepdims=True),
                 jnp.zeros((7, EMB), jnp.float32)], 0)[None]

            @pl.when(j == 0)
            def _():
                p_ref[...] = p

            @pl.when(j > 0)
            def _():
                p_ref[...] += p

    out_specs = [pl.BlockSpec((CW, EMB), lambda b, j: (b * NBLK + j, 0))]
    out_shape = [jax.ShapeDtypeStruct((B * NPAD, EMB), jnp.float32)]
    if last:
        out_specs.append(pl.BlockSpec((1, 8, EMB), lambda b, j: (b, 0, 0)))
        out_shape.append(jax.ShapeDtypeStruct((B, 8, EMB), jnp.float32))
    res = pl.pallas_call(
        body,
        grid=(B, N // CW),
        in_specs=[
            pl.BlockSpec((1, CW, H2), lambda b, j: (b, j, 0)),
            pl.BlockSpec((1, 8, H2), lambda b, j: (b, 0, 0)),
            pl.BlockSpec((1, H2), lambda b, j: (0, 0)),
            pl.BlockSpec((1, H2), lambda b, j: (0, 0)),
            pl.BlockSpec((H2, EMB), lambda b, j: (0, 0)),
            pl.BlockSpec((1, EMB), lambda b, j: (0, 0)),
        ],
        out_specs=out_specs,
        out_shape=out_shape,
    )(t, stats, gamma_row, beta_row, W2_l, b2_row)
    return res if last else (res[0], None)


def _head_call(pooled, subj, W_ext, b_ext_row, W_pred, b_pred_row):
    def body(p_ref, s_ref, we_ref, be_ref, wp_ref, bp_ref, o_ref, l1_ref):
        pm = p_ref[:, 0, :] * (1.0 / N)
        ext = s_ref[...] @ we_ref[...] + be_ref[...]
        cat = jnp.concatenate([pm, ext], 1)
        o_ref[...] = cat @ wp_ref[...] + bp_ref[...]
        l1_ref[...] = jnp.mean(jnp.abs(wp_ref[...]))[None, None]

    return pl.pallas_call(
        body,
        out_shape=[
            jax.ShapeDtypeStruct((B, OUT), jnp.float32),
            jax.ShapeDtypeStruct((1, 1), jnp.float32),
        ],
    )(pooled, subj, W_ext, b_ext_row, W_pred, b_pred_row)


def kernel(node_features, edge_index, edge_attr, subject_features, W_in, b_in,
           W_ee, b_ee, W1, b1, gamma, beta, W2, b2, W_ext, b_ext, W_pred,
           b_pred):
    src = edge_index[:, 0, :]
    dst = edge_index[:, 1, :]
    offs = (jnp.arange(B, dtype=jnp.int32) * NPAD)[:, None]
    src_g = (src + offs).reshape(B * NT, NCHUNK, CW)
    dst_l = dst.reshape(B * NT, NCHUNK, CW)
    a = edge_attr
    rows = jnp.concatenate(
        [a, jnp.ones_like(a), jnp.zeros((B, E, SW - 2), jnp.float32)], axis=2)
    rows = rows.reshape(B * NT * NCHUNK, CW, SW)

    S_flat = _sdeg(rows, dst_l)
    Wc = _wc_call(W_ee, b_ee, W1)

    nf_pad = jnp.pad(node_features, ((0, 0), (0, 0), (0, 3)))
    Win_pad = jnp.pad(W_in, ((0, 3), (0, 0)))
    x = _x0_call(nf_pad, Win_pad, b_in.reshape(1, EMB))

    pooled = None
    for l in range(3):
        aggr = _aggr(x, src_g, dst_l)
        t, stats = _tstats_call(aggr, S_flat, W1[l, :EMB, :], Wc[l],
                                b1[l].reshape(1, H2))
        last = l == 2
        x, pooled = _update_call(t, stats, gamma[l].reshape(1, H2),
                                 beta[l].reshape(1, H2), W2[l],
                                 b2[l].reshape(1, EMB), last)

    out, l1 = _head_call(pooled, subject_features, W_ext,
                         b_ext.reshape(1, EMB), W_pred,
                         b_pred.reshape(1, EMB))
    return out, l1.reshape(())


# speedup vs baseline: 7.5844x; 2.1792x over previous
import functools

import jax
import jax.numpy as jnp
from jax import lax
from jax.experimental import pallas as pl
from jax.experimental.pallas import tpu as pltpu
from jax.experimental.pallas import tpu_sc as plsc

B = 2
N = 10000
E = 160000
EMB = 128
H2 = 256
OUT = 128
EXT = 64
NPAD = 10240
NT = 16
NCHUNK = 125
CW = 80
SW = 16
RPT = NPAD // NT
RBK = 2048
NRB = NPAD // RBK


def _sc_mesh():
    return plsc.VectorSubcoreMesh(core_axis_name="c", subcore_axis_name="s")


def _gs_body(D, x_hbm, src_hbm, dst_hbm, out_hbm, si0, si1, dst_v, buf0, buf1,
             acc, semi0, semi1, semg0, semg1):
    c = lax.axis_index("c")
    s = lax.axis_index("s")
    w = c * NT + s
    z = jnp.zeros((16,), jnp.float32)

    def zrow(i, carry):
        for k in range(D // 16):
            buf0[i, pl.ds(k * 16, 16)] = z
        return carry

    lax.fori_loop(0, CW, zrow, 0)
    for k in range(RPT // CW):
        pltpu.sync_copy(buf0, acc.at[pl.ds(s * RPT + k * CW, CW)])
    pltpu.sync_copy(src_hbm.at[w, 0], si0)
    pltpu.sync_copy(src_hbm.at[w, 1], si1)
    pltpu.sync_copy(dst_hbm.at[w], dst_v)
    pltpu.async_copy(x_hbm.at[si0], buf0, semg0)
    plsc.subcore_barrier()

    def body(j, carry):
        pltpu.make_async_copy(x_hbm.at[si0], buf0, semg0).wait()
        pltpu.async_copy(src_hbm.at[w, 2 * j + 2], si0, semi0)
        pltpu.async_copy(x_hbm.at[si1], buf1, semg1)
        pltpu.sync_copy(buf0, acc.at[dst_v.at[2 * j]], add=True)
        pltpu.make_async_copy(x_hbm.at[si1], buf1, semg1).wait()
        pltpu.make_async_copy(src_hbm.at[w, 0], si0, semi0).wait()
        pltpu.async_copy(src_hbm.at[w, 2 * j + 3], si1, semi1)
        pltpu.async_copy(x_hbm.at[si0], buf0, semg0)
        pltpu.sync_copy(buf1, acc.at[dst_v.at[2 * j + 1]], add=True)
        pltpu.make_async_copy(src_hbm.at[w, 0], si1, semi1).wait()
        return carry

    lax.fori_loop(0, (NCHUNK - 1) // 2, body, 0)
    pltpu.make_async_copy(x_hbm.at[si0], buf0, semg0).wait()
    pltpu.sync_copy(buf0, acc.at[dst_v.at[NCHUNK - 1]], add=True)
    plsc.subcore_barrier()
    pltpu.sync_copy(acc.at[pl.ds(s * RPT, RPT)],
                    out_hbm.at[pl.ds(c * NPAD + s * RPT, RPT)])


def _gs(x_flat, src_g, dst_l, D):
    f = functools.partial(
        pl.kernel,
        out_type=jax.ShapeDtypeStruct((B * NPAD, D), jnp.float32),
        mesh=_sc_mesh(),
        scratch_types=[
            pltpu.VMEM((CW,), jnp.int32),
            pltpu.VMEM((CW,), jnp.int32),
            pltpu.VMEM((NCHUNK, CW), jnp.int32),
            pltpu.VMEM((CW, D), jnp.float32),
            pltpu.VMEM((CW, D), jnp.float32),
            pltpu.VMEM_SHARED((NPAD, D), jnp.float32),
            pltpu.SemaphoreType.DMA,
            pltpu.SemaphoreType.DMA,
            pltpu.SemaphoreType.DMA,
            pltpu.SemaphoreType.DMA,
        ],
        compiler_params=pltpu.CompilerParams(use_tc_tiling_on_sc=False),
    )(functools.partial(_gs_body, D))
    return f(x_flat, src_g, dst_l)


def _wc_call(W_ee, b_ee, W1):
    def body(we_ref, be_ref, w1_ref, o_ref):
        w1b = w1_ref[0, EMB:, :]
        r0 = we_ref[0] @ w1b
        r1 = be_ref[0] @ w1b
        o_ref[...] = jnp.concatenate(
            [r0, r1, jnp.zeros((SW - 2, H2), jnp.float32)], 0)[None]

    return pl.pallas_call(
        body,
        grid=(3,),
        in_specs=[
            pl.BlockSpec((1, 1, EMB), lambda l: (l, 0, 0)),
            pl.BlockSpec((1, 1, EMB), lambda l: (l, 0, 0)),
            pl.BlockSpec((1, 2 * EMB, H2), lambda l: (l, 0, 0)),
        ],
        out_specs=pl.BlockSpec((1, SW, H2), lambda l: (l, 0, 0)),
        out_shape=jax.ShapeDtypeStruct((3, SW, H2), jnp.float32),
    )(W_ee, b_ee.reshape(3, 1, EMB), W1)


def _x0_call(nf_pad, W_in_pad, b_in_row):
    def body(nf_ref, w_ref, b_ref, o_ref):
        o_ref[...] = (nf_ref[0] @ w_ref[...] + b_ref[...])[None]

    return pl.pallas_call(
        body,
        grid=(B, NRB),
        in_specs=[
            pl.BlockSpec((1, RBK, 24), lambda b, j: (b, j, 0)),
            pl.BlockSpec((24, EMB), lambda b, j: (0, 0)),
            pl.BlockSpec((1, EMB), lambda b, j: (0, 0)),
        ],
        out_specs=pl.BlockSpec((1, RBK, EMB), lambda b, j: (b, j, 0)),
        out_shape=jax.ShapeDtypeStruct((B, NPAD, EMB), jnp.float32),
    )(nf_pad, W_in_pad, b_in_row)


def _colsum_call(aggr3, S3):
    def body(a_ref, s_ref, oa_ref, os_ref):
        j = pl.program_id(1)
        pa = jnp.concatenate(
            [jnp.sum(a_ref[0], 0, keepdims=True),
             jnp.zeros((7, EMB), jnp.float32)], 0)[None]
        ps = jnp.concatenate(
            [jnp.sum(s_ref[0], 0, keepdims=True),
             jnp.zeros((7, SW), jnp.float32)], 0)[None]

        @pl.when(j == 0)
        def _():
            oa_ref[...] = pa
            os_ref[...] = ps

        @pl.when(j > 0)
        def _():
            oa_ref[...] += pa
            os_ref[...] += ps

    return pl.pallas_call(
        body,
        grid=(B, NRB),
        in_specs=[
            pl.BlockSpec((1, RBK, EMB), lambda b, j: (b, j, 0)),
            pl.BlockSpec((1, RBK, SW), lambda b, j: (b, j, 0)),
        ],
        out_specs=[
            pl.BlockSpec((1, 8, EMB), lambda b, j: (b, 0, 0)),
            pl.BlockSpec((1, 8, SW), lambda b, j: (b, 0, 0)),
        ],
        out_shape=[
            jax.ShapeDtypeStruct((B, 8, EMB), jnp.float32),
            jax.ShapeDtypeStruct((B, 8, SW), jnp.float32),
        ],
    )(aggr3, S3)


def _tstats_call(aggr3, S3, asum, ssum, W1t, Wc_l, b1_row):
    def body(a_ref, s_ref, as_ref, ss_ref, w_ref, wc_ref, b_ref, t_ref,
             st_ref):
        j = pl.program_id(1)
        mu = (as_ref[0, 0:1, :] @ w_ref[...]
              + ss_ref[0, 0:1, :] @ wc_ref[...]) * (1.0 / N) + b_ref[...]
        t = a_ref[0] @ w_ref[...] + s_ref[0] @ wc_ref[...] + b_ref[...]
        t_ref[...] = t[None]
        rid = lax.broadcasted_iota(jnp.int32, (RBK, 1), 0)
        m = rid < (N - j * RBK)
        tc = jnp.where(m, t - mu, 0.0)
        sq = jnp.sum(tc * tc, 0, keepdims=True)

        @pl.when(j == 0)
        def _():
            st_ref[...] = jnp.concatenate(
                [sq, mu, jnp.zeros((6, H2), jnp.float32)], 0)[None]

        @pl.when(j > 0)
        def _():
            st_ref[...] += jnp.concatenate(
                [sq, jnp.zeros((7, H2), jnp.float32)], 0)[None]

    return pl.pallas_call(
        body,
        grid=(B, NRB),
        in_specs=[
            pl.BlockSpec((1, RBK, EMB), lambda b, j: (b, j, 0)),
            pl.BlockSpec((1, RBK, SW), lambda b, j: (b, j, 0)),
            pl.BlockSpec((1, 8, EMB), lambda b, j: (b, 0, 0)),
            pl.BlockSpec((1, 8, SW), lambda b, j: (b, 0, 0)),
            pl.BlockSpec((EMB, H2), lambda b, j: (0, 0)),
            pl.BlockSpec((SW, H2), lambda b, j: (0, 0)),
            pl.BlockSpec((1, H2), lambda b, j: (0, 0)),
        ],
        out_specs=[
            pl.BlockSpec((1, RBK, H2), lambda b, j: (b, j, 0)),
            pl.BlockSpec((1, 8, H2), lambda b, j: (b, 0, 0)),
        ],
        out_shape=[
            jax.ShapeDtypeStruct((B, NPAD, H2), jnp.float32),
            jax.ShapeDtypeStruct((B, 8, H2), jnp.float32),
        ],
    )(aggr3, S3, asum, ssum, W1t, Wc_l, b1_row)


def _update_call(t, stats, gamma_row, beta_row, W2_l, b2_row, last):
    def body(t_ref, st_ref, g_ref, be_ref, w_ref, b_ref, x_ref, *rest):
        var = st_ref[0, 0, :] * (1.0 / N)
        mean = st_ref[0, 1, :]
        inv = lax.rsqrt(var + 1e-5)
        tn = (t_ref[0] - mean) * (inv * g_ref[0]) + be_ref[0]
        r = jnp.maximum(tn, 0.0)
        hb = r @ w_ref[...] + b_ref[...]
        if not last:
            hb = jnp.maximum(hb, 0.0)
        x_ref[...] = hb[None]
        if last:
            p_ref = rest[0]
            j = pl.program_id(1)
            rid = lax.broadcasted_iota(jnp.int32, (RBK, 1), 0)
            m = rid < (N - j * RBK)
            hm = jnp.where(m, hb, 0.0)
            p = jnp.concatenate(
                [jnp.sum(hm, 0, keepdims=True),
                 jnp.zeros((7, EMB), jnp.float32)], 0)[None]

            @pl.when(j == 0)
            def _():
                p_ref[...] = p

            @pl.when(j > 0)
            def _():
                p_ref[...] += p

    out_specs = [pl.BlockSpec((1, RBK, EMB), lambda b, j: (b, j, 0))]
    out_shape = [jax.ShapeDtypeStruct((B, NPAD, EMB), jnp.float32)]
    if last:
        out_specs.append(pl.BlockSpec((1, 8, EMB), lambda b, j: (b, 0, 0)))
        out_shape.append(jax.ShapeDtypeStruct((B, 8, EMB), jnp.float32))
    res = pl.pallas_call(
        body,
        grid=(B, NRB),
        in_specs=[
            pl.BlockSpec((1, RBK, H2), lambda b, j: (b, j, 0)),
            pl.BlockSpec((1, 8, H2), lambda b, j: (b, 0, 0)),
            pl.BlockSpec((1, H2), lambda b, j: (0, 0)),
            pl.BlockSpec((1, H2), lambda b, j: (0, 0)),
            pl.BlockSpec((H2, EMB), lambda b, j: (0, 0)),
            pl.BlockSpec((1, EMB), lambda b, j: (0, 0)),
        ],
        out_specs=out_specs,
        out_shape=out_shape,
    )(t, stats, gamma_row, beta_row, W2_l, b2_row)
    return res if last else (res[0], None)


def _head_call(pooled, subj, W_ext, b_ext_row, W_pred, b_pred_row):
    def body(p_ref, s_ref, we_ref, be_ref, wp_ref, bp_ref, o_ref, l1_ref):
        pm = p_ref[:, 0, :] * (1.0 / N)
        ext = s_ref[...] @ we_ref[...] + be_ref[...]
        cat = jnp.concatenate([pm, ext], 1)
        o_ref[...] = cat @ wp_ref[...] + bp_ref[...]
        l1_ref[...] = jnp.mean(jnp.abs(wp_ref[...]))[None, None]

    return pl.pallas_call(
        body,
        out_shape=[
            jax.ShapeDtypeStruct((B, OUT), jnp.float32),
            jax.ShapeDtypeStruct((1, 1), jnp.float32),
        ],
    )(pooled, subj, W_ext, b_ext_row, W_pred, b_pred_row)


def kernel(node_features, edge_index, edge_attr, subject_features, W_in, b_in,
           W_ee, b_ee, W1, b1, gamma, beta, W2, b2, W_ext, b_ext, W_pred,
           b_pred):
    src = edge_index[:, 0, :]
    dst = edge_index[:, 1, :]
    offs = (jnp.arange(B, dtype=jnp.int32) * NPAD)[:, None]
    src_g = (src + offs).reshape(B * NT, NCHUNK, CW)
    src_g = jnp.pad(src_g, ((0, 0), (0, 1), (0, 0)))
    dst_l = dst.reshape(B * NT, NCHUNK, CW)
    a = edge_attr
    rows = jnp.concatenate(
        [a, jnp.ones_like(a), jnp.zeros((B, E, SW - 2), jnp.float32)], axis=2)
    rows_flat = rows.reshape(B * E, SW)
    eid = jnp.arange(B * E, dtype=jnp.int32).reshape(B * NT, NCHUNK, CW)
    eid = jnp.pad(eid, ((0, 0), (0, 1), (0, 0)))

    S_flat = _gs(rows_flat, eid, dst_l, SW)
    S3 = S_flat.reshape(B, NPAD, SW)
    Wc = _wc_call(W_ee, b_ee, W1)

    nf_pad = jnp.pad(node_features,
                     ((0, 0), (0, NPAD - N), (0, 24 - node_features.shape[2])))
    Win_pad = jnp.pad(W_in, ((0, 3), (0, 0)))
    x = _x0_call(nf_pad, Win_pad, b_in.reshape(1, EMB))

    pooled = None
    for l in range(3):
        aggr_flat = _gs(x.reshape(B * NPAD, EMB), src_g, dst_l, EMB)
        aggr3 = aggr_flat.reshape(B, NPAD, EMB)
        asum, ssum = _colsum_call(aggr3, S3)
        t, stats = _tstats_call(aggr3, S3, asum, ssum, W1[l, :EMB, :], Wc[l],
                                b1[l].reshape(1, H2))
        last = l == 2
        x, pooled = _update_call(t, stats, gamma[l].reshape(1, H2),
                                 beta[l].reshape(1, H2), W2[l],
                                 b2[l].reshape(1, EMB), last)

    out, l1 = _head_call(pooled, subject_features, W_ext,
                         b_ext.reshape(1, EMB), W_pred,
                         b_pred.reshape(1, EMB))
    return out, l1.reshape(())
